# Initial kernel scaffold; baseline (speedup 1.0000x reference)
import jax
import jax.numpy as jnp
from jax.experimental import pallas as pl


def kernel(x, edge_attr, edge_index, W1, b1, W2, b2):
    raise NotImplementedError("write your pallas kernel here")


# speedup vs baseline: 31.9704x; 31.9704x over previous
import functools

import jax
import jax.numpy as jnp
from jax import lax
from jax.experimental import pallas as pl
from jax.experimental.pallas import tpu as pltpu
from jax.experimental.pallas import tpu_sc as plsc

N = 10000
E = 320000
HID = 16
NPAD = 10240
NC = 2
NS = 16
NW = NC * NS
CH = 80
EPW = E // NW
NCH = EPW // CH
ROWS = E // CH
SPW = NPAD // NS

_f32 = jnp.float32
_SDS = jax.ShapeDtypeStruct


def _mesh():
    return plsc.VectorSubcoreMesh(core_axis_name="c", subcore_axis_name="s")


_SC_PARAMS = pltpu.CompilerParams(use_tc_tiling_on_sc=False)


def _sc_pass_a(row2d, col2d, attr2d, ones_ch, zrow):
    @functools.partial(
        pl.kernel,
        out_type=(_SDS((NC, NPAD), _f32), _SDS((NC, NPAD), _f32)),
        mesh=_mesh(),
        compiler_params=_SC_PARAMS,
        scratch_types=[
            pltpu.VMEM((NCH, CH), jnp.int32),
            pltpu.VMEM((NCH, CH), jnp.int32),
            pltpu.VMEM((NCH, CH), _f32),
            pltpu.VMEM((CH,), _f32),
            pltpu.VMEM_SHARED((NPAD,), _f32),
            pltpu.VMEM_SHARED((NPAD,), _f32),
        ],
    )
    def k(row_hbm, col_hbm, attr_hbm, ones_hbm, z_hbm,
          outdeg_hbm, outxe_hbm,
          rowbuf, colbuf, attrbuf, onesbuf, deg_acc, xe_acc):
        c = lax.axis_index("c")
        s = lax.axis_index("s")
        wid = c * NS + s
        sl = pl.ds(s * SPW, SPW)
        pltpu.sync_copy(z_hbm, deg_acc.at[sl])
        pltpu.sync_copy(z_hbm, xe_acc.at[sl])
        pltpu.sync_copy(ones_hbm, onesbuf)
        pltpu.sync_copy(row_hbm.at[wid], rowbuf)
        pltpu.sync_copy(col_hbm.at[wid], colbuf)
        pltpu.sync_copy(attr_hbm.at[wid], attrbuf)
        plsc.subcore_barrier()

        def step(j, carry):
            pltpu.sync_copy(onesbuf, deg_acc.at[colbuf.at[j]], add=True)
            pltpu.sync_copy(attrbuf.at[j], xe_acc.at[rowbuf.at[j]], add=True)
            return carry

        lax.fori_loop(0, NCH, step, 0)
        plsc.subcore_barrier()
        pltpu.sync_copy(deg_acc.at[sl], outdeg_hbm.at[c, sl])
        pltpu.sync_copy(xe_acc.at[sl], outxe_hbm.at[c, sl])

    return k(row2d, col2d, attr2d, ones_ch, zrow)


def _sc_pass_c(row2d, col2d, g1, zrow16):
    @functools.partial(
        pl.kernel,
        out_type=_SDS((NC, NPAD, HID), _f32),
        mesh=_mesh(),
        compiler_params=_SC_PARAMS,
        scratch_types=[
            pltpu.VMEM((NCH, CH), jnp.int32),
            pltpu.VMEM((NCH, CH), jnp.int32),
            pltpu.VMEM((CH, HID), _f32),
            pltpu.VMEM_SHARED((NPAD, HID), _f32),
            pltpu.SemaphoreType.DMA,
        ],
    )
    def k(row_hbm, col_hbm, g1_hbm, z_hbm, out_hbm,
          rowbuf, colbuf, vals, acc, sem):
        c = lax.axis_index("c")
        s = lax.axis_index("s")
        wid = c * NS + s
        sl = pl.ds(s * SPW, SPW)
        pltpu.sync_copy(z_hbm, acc.at[sl, :])
        pltpu.sync_copy(row_hbm.at[wid], rowbuf)
        pltpu.sync_copy(col_hbm.at[wid], colbuf)
        plsc.subcore_barrier()

        def step(j, carry):
            pltpu.async_copy(g1_hbm.at[rowbuf.at[j]], vals, sem).wait()
            pltpu.sync_copy(vals, acc.at[colbuf.at[j]], add=True)
            return carry

        lax.fori_loop(0, NCH, step, 0)
        plsc.subcore_barrier()
        pltpu.sync_copy(acc.at[sl, :], out_hbm.at[c, sl, :])

    return k(row2d, col2d, g1, zrow16)


def _sc_pass_e(row2d, col2d, g2, zrow):
    @functools.partial(
        pl.kernel,
        out_type=_SDS((NC, NPAD), _f32),
        mesh=_mesh(),
        compiler_params=_SC_PARAMS,
        scratch_types=[
            pltpu.VMEM((NCH, CH), jnp.int32),
            pltpu.VMEM((NCH, CH), jnp.int32),
            pltpu.VMEM((CH,), _f32),
            pltpu.VMEM_SHARED((N,), _f32),
            pltpu.VMEM_SHARED((NPAD,), _f32),
            pltpu.SemaphoreType.DMA,
        ],
    )
    def k(row_hbm, col_hbm, g2_hbm, z_hbm, out_hbm,
          rowbuf, colbuf, vals, g2s, acc, sem):
        c = lax.axis_index("c")
        s = lax.axis_index("s")
        wid = c * NS + s
        sl = pl.ds(s * SPW, SPW)
        pltpu.sync_copy(z_hbm, acc.at[sl])

        @pl.when(s == 0)
        def _stage():
            pltpu.sync_copy(g2_hbm, g2s)

        pltpu.sync_copy(row_hbm.at[wid], rowbuf)
        pltpu.sync_copy(col_hbm.at[wid], colbuf)
        plsc.subcore_barrier()

        def step(j, carry):
            pltpu.async_copy(g2s.at[rowbuf.at[j]], vals, sem).wait()
            pltpu.sync_copy(vals, acc.at[colbuf.at[j]], add=True)
            return carry

        lax.fori_loop(0, NCH, step, 0)
        plsc.subcore_barrier()
        pltpu.sync_copy(acc.at[sl], out_hbm.at[c, sl])

    return k(row2d, col2d, g2, zrow)


def _tc_pass_b(x, deg0, deg1, xe0, xe1, W1):
    def body(x_ref, d0, d1, xa, xb, w1, g1_ref, dinv_ref):
        deg = 1.0 + d0[...] + d1[...]
        dinv = lax.rsqrt(deg)
        xe = xa[...] + xb[...]
        h1 = jnp.dot(x_ref[...], w1[0:128, :], preferred_element_type=_f32)
        h1 = h1 + xe[:, None] * w1[128:129, :]
        g1_ref[...] = h1 * dinv[:, None]
        dinv_ref[...] = dinv

    return pl.pallas_call(
        body,
        out_shape=(_SDS((N, HID), _f32), _SDS((N,), _f32)),
    )(x, deg0, deg1, xe0, xe1, W1)


def _tc_pass_d(s1a, s1b, g1, dinv, b1, W2):
    def body(sa, sb, g1r, dv, b1r, w2, g2_ref):
        S = sa[...] + sb[...] + g1r[...]
        out1 = dv[...][:, None] * S + b1r[...][None, :]
        h = jnp.maximum(out1, 0.0)
        h2 = jnp.dot(h, w2[...], preferred_element_type=_f32)
        g2_ref[...] = h2[:, 0] * dv[...]

    return pl.pallas_call(
        body,
        out_shape=_SDS((N,), _f32),
    )(s1a, s1b, g1, dinv, b1, W2)


def _tc_pass_f(s2a, s2b, g2, dinv, b2):
    def body(sa, sb, g2r, dv, b2r, out_ref):
        v = dv[...] * (sa[...] + sb[...] + g2r[...])
        out_ref[...] = v[:, None] + b2r[...][None, :]

    return pl.pallas_call(
        body,
        out_shape=_SDS((N, 1), _f32),
    )(s2a, s2b, g2, dinv, b2)


def kernel(x, edge_attr, edge_index, W1, b1, W2, b2):
    row = edge_index[0].reshape(NW, NCH, CH)
    col = edge_index[1].reshape(NW, NCH, CH)
    attr = edge_attr[:, 0].reshape(NW, NCH, CH)
    ones_ch = jnp.ones((CH,), _f32)
    zrow = jnp.zeros((SPW,), _f32)
    zrow16 = jnp.zeros((SPW, HID), _f32)

    degp, xep = _sc_pass_a(row, col, attr, ones_ch, zrow)
    g1, dinv = _tc_pass_b(x, degp[0, :N], degp[1, :N], xep[0, :N], xep[1, :N], W1)
    s1p = _sc_pass_c(row, col, g1, zrow16)
    g2 = _tc_pass_d(s1p[0, :N, :], s1p[1, :N, :], g1, dinv, b1, W2)
    s2p = _sc_pass_e(row, col, g2, zrow)
    return _tc_pass_f(s2p[0, :N], s2p[1, :N], g2, dinv, b2)


# speedup vs baseline: 56.4769x; 1.7665x over previous
import functools

import jax
import jax.numpy as jnp
from jax import lax
from jax.experimental import pallas as pl
from jax.experimental.pallas import tpu as pltpu
from jax.experimental.pallas import tpu_sc as plsc

N = 10000
E = 320000
HID = 16
NPAD = 10240
NC = 2
NS = 16
NW = NC * NS
CH = 80
EPW = E // NW
NCH = EPW // CH
SPW = NPAD // NS
NB = 25
NGRP = NCH // NB

_f32 = jnp.float32
_SDS = jax.ShapeDtypeStruct


def _mesh():
    return plsc.VectorSubcoreMesh(core_axis_name="c", subcore_axis_name="s")


_SC_PARAMS = pltpu.CompilerParams(use_tc_tiling_on_sc=False)


def _sc_pass_a(row3d, col3d, attr3d, ones_ch, zrow):
    @functools.partial(
        pl.kernel,
        out_type=(_SDS((NC, NPAD), _f32), _SDS((NC, NPAD), _f32)),
        mesh=_mesh(),
        compiler_params=_SC_PARAMS,
        scratch_types=[
            pltpu.VMEM((NCH, CH), jnp.int32),
            pltpu.VMEM((NCH, CH), jnp.int32),
            pltpu.VMEM((NCH, CH), _f32),
            pltpu.VMEM((CH,), _f32),
            pltpu.VMEM_SHARED((NPAD,), _f32),
            pltpu.VMEM_SHARED((NPAD,), _f32),
            pltpu.SemaphoreType.DMA,
        ],
    )
    def k(row_hbm, col_hbm, attr_hbm, ones_hbm, z_hbm,
          outdeg_hbm, outxe_hbm,
          rowbuf, colbuf, attrbuf, onesbuf, deg_acc, xe_acc, sem):
        c = lax.axis_index("c")
        s = lax.axis_index("s")
        wid = c * NS + s
        sl = pl.ds(s * SPW, SPW)
        pltpu.sync_copy(z_hbm, deg_acc.at[sl])
        pltpu.sync_copy(z_hbm, xe_acc.at[sl])
        pltpu.sync_copy(ones_hbm, onesbuf)
        pltpu.sync_copy(row_hbm.at[wid], rowbuf)
        pltpu.sync_copy(col_hbm.at[wid], colbuf)
        pltpu.sync_copy(attr_hbm.at[wid], attrbuf)
        plsc.subcore_barrier()

        def fire(j, carry):
            pltpu.async_copy(onesbuf, deg_acc.at[colbuf.at[j]], sem, add=True)
            pltpu.async_copy(attrbuf.at[j], xe_acc.at[rowbuf.at[j]], sem,
                             add=True)
            return carry

        lax.fori_loop(0, NCH, fire, 0)

        def drain(j, carry):
            pltpu.make_async_copy(onesbuf, deg_acc.at[colbuf.at[j]], sem).wait()
            pltpu.make_async_copy(attrbuf.at[j], xe_acc.at[rowbuf.at[j]],
                                  sem).wait()
            return carry

        lax.fori_loop(0, NCH, drain, 0)
        plsc.subcore_barrier()
        pltpu.sync_copy(deg_acc.at[sl], outdeg_hbm.at[c, sl])
        pltpu.sync_copy(xe_acc.at[sl], outxe_hbm.at[c, sl])

    return k(row3d, col3d, attr3d, ones_ch, zrow)


def _sc_pass_c(row3d, col3d, g1, zrow16):
    @functools.partial(
        pl.kernel,
        out_type=_SDS((NC, NPAD, HID), _f32),
        mesh=_mesh(),
        compiler_params=_SC_PARAMS,
        scratch_types=[
            pltpu.VMEM((NCH, CH), jnp.int32),
            pltpu.VMEM((NCH, CH), jnp.int32),
            pltpu.VMEM((2 * NB, CH, HID), _f32),
            pltpu.VMEM_SHARED((NPAD, HID), _f32),
            pltpu.SemaphoreType.DMA,
            pltpu.SemaphoreType.DMA,
        ],
    )
    def k(row_hbm, col_hbm, g1_hbm, z_hbm, out_hbm,
          rowbuf, colbuf, vals, acc, gsem, ssem):
        c = lax.axis_index("c")
        s = lax.axis_index("s")
        wid = c * NS + s
        sl = pl.ds(s * SPW, SPW)
        pltpu.sync_copy(z_hbm, acc.at[sl, :])
        pltpu.sync_copy(row_hbm.at[wid], rowbuf)
        pltpu.sync_copy(col_hbm.at[wid], colbuf)
        plsc.subcore_barrier()

        def fire_gathers(g, bank):
            def fg(i, carry):
                pltpu.async_copy(g1_hbm.at[rowbuf.at[g * NB + i]],
                                 vals.at[bank * NB + i], gsem)
                return carry
            lax.fori_loop(0, NB, fg, 0)

        def drain_gathers(g, bank):
            def dg(i, carry):
                pltpu.make_async_copy(g1_hbm.at[rowbuf.at[g * NB + i]],
                                      vals.at[bank * NB + i], gsem).wait()
                return carry
            lax.fori_loop(0, NB, dg, 0)

        def fire_scatters(g, bank):
            def fs(i, carry):
                pltpu.async_copy(vals.at[bank * NB + i],
                                 acc.at[colbuf.at[g * NB + i]], ssem, add=True)
                return carry
            lax.fori_loop(0, NB, fs, 0)

        def drain_scatters(g, bank):
            def ds(i, carry):
                pltpu.make_async_copy(vals.at[bank * NB + i],
                                      acc.at[colbuf.at[g * NB + i]],
                                      ssem).wait()
                return carry
            lax.fori_loop(0, NB, ds, 0)

        fire_gathers(0, 0)

        def grp(g, carry):
            bank = lax.rem(g, 2)
            drain_gathers(g, bank)

            @pl.when(g < NGRP - 1)
            def _():
                fire_gathers(g + 1, 1 - bank)

            @pl.when(g > 0)
            def _():
                drain_scatters(g - 1, 1 - bank)

            fire_scatters(g, bank)
            return carry

        lax.fori_loop(0, NGRP, grp, 0)
        drain_scatters(NGRP - 1, lax.rem(NGRP - 1, 2))
        plsc.subcore_barrier()
        pltpu.sync_copy(acc.at[sl, :], out_hbm.at[c, sl, :])

    return k(row3d, col3d, g1, zrow16)


def _sc_pass_e(row3d, col3d, g2, zrow):
    @functools.partial(
        pl.kernel,
        out_type=_SDS((NC, NPAD), _f32),
        mesh=_mesh(),
        compiler_params=_SC_PARAMS,
        scratch_types=[
            pltpu.VMEM((NCH, CH), jnp.int32),
            pltpu.VMEM((NCH, CH), jnp.int32),
            pltpu.VMEM((2 * NB, CH), _f32),
            pltpu.VMEM_SHARED((N,), _f32),
            pltpu.VMEM_SHARED((NPAD,), _f32),
            pltpu.SemaphoreType.DMA,
            pltpu.SemaphoreType.DMA,
        ],
    )
    def k(row_hbm, col_hbm, g2_hbm, z_hbm, out_hbm,
          rowbuf, colbuf, vals, g2s, acc, gsem, ssem):
        c = lax.axis_index("c")
        s = lax.axis_index("s")
        wid = c * NS + s
        sl = pl.ds(s * SPW, SPW)
        pltpu.sync_copy(z_hbm, acc.at[sl])

        @pl.when(s == 0)
        def _stage():
            pltpu.sync_copy(g2_hbm, g2s)

        pltpu.sync_copy(row_hbm.at[wid], rowbuf)
        pltpu.sync_copy(col_hbm.at[wid], colbuf)
        plsc.subcore_barrier()

        def fire_gathers(g, bank):
            def fg(i, carry):
                pltpu.async_copy(g2s.at[rowbuf.at[g * NB + i]],
                                 vals.at[bank * NB + i], gsem)
                return carry
            lax.fori_loop(0, NB, fg, 0)

        def drain_gathers(g, bank):
            def dg(i, carry):
                pltpu.make_async_copy(g2s.at[rowbuf.at[g * NB + i]],
                                      vals.at[bank * NB + i], gsem).wait()
                return carry
            lax.fori_loop(0, NB, dg, 0)

        def fire_scatters(g, bank):
            def fs(i, carry):
                pltpu.async_copy(vals.at[bank * NB + i],
                                 acc.at[colbuf.at[g * NB + i]], ssem, add=True)
                return carry
            lax.fori_loop(0, NB, fs, 0)

        def drain_scatters(g, bank):
            def ds(i, carry):
                pltpu.make_async_copy(vals.at[bank * NB + i],
                                      acc.at[colbuf.at[g * NB + i]],
                                      ssem).wait()
                return carry
            lax.fori_loop(0, NB, ds, 0)

        fire_gathers(0, 0)

        def grp(g, carry):
            bank = lax.rem(g, 2)
            drain_gathers(g, bank)

            @pl.when(g < NGRP - 1)
            def _():
                fire_gathers(g + 1, 1 - bank)

            @pl.when(g > 0)
            def _():
                drain_scatters(g - 1, 1 - bank)

            fire_scatters(g, bank)
            return carry

        lax.fori_loop(0, NGRP, grp, 0)
        drain_scatters(NGRP - 1, lax.rem(NGRP - 1, 2))
        plsc.subcore_barrier()
        pltpu.sync_copy(acc.at[sl], out_hbm.at[c, sl])

    return k(row3d, col3d, g2, zrow)


def _tc_pass_b(x, deg0, deg1, xe0, xe1, W1):
    def body(x_ref, d0, d1, xa, xb, w1, g1_ref, dinv_ref):
        deg = 1.0 + d0[...] + d1[...]
        dinv = lax.rsqrt(deg)
        xe = xa[...] + xb[...]
        h1 = jnp.dot(x_ref[...], w1[0:128, :], preferred_element_type=_f32)
        h1 = h1 + xe[:, None] * w1[128:129, :]
        g1_ref[...] = h1 * dinv[:, None]
        dinv_ref[...] = dinv

    return pl.pallas_call(
        body,
        out_shape=(_SDS((N, HID), _f32), _SDS((N,), _f32)),
    )(x, deg0, deg1, xe0, xe1, W1)


def _tc_pass_d(s1a, s1b, g1, dinv, b1, W2):
    def body(sa, sb, g1r, dv, b1r, w2, g2_ref):
        S = sa[...] + sb[...] + g1r[...]
        out1 = dv[...][:, None] * S + b1r[...][None, :]
        h = jnp.maximum(out1, 0.0)
        h2 = jnp.dot(h, w2[...], preferred_element_type=_f32)
        g2_ref[...] = h2[:, 0] * dv[...]

    return pl.pallas_call(
        body,
        out_shape=_SDS((N,), _f32),
    )(s1a, s1b, g1, dinv, b1, W2)


def _tc_pass_f(s2a, s2b, g2, dinv, b2):
    def body(sa, sb, g2r, dv, b2r, out_ref):
        v = dv[...] * (sa[...] + sb[...] + g2r[...])
        out_ref[...] = v[:, None] + b2r[...][None, :]

    return pl.pallas_call(
        body,
        out_shape=_SDS((N, 1), _f32),
    )(s2a, s2b, g2, dinv, b2)


def kernel(x, edge_attr, edge_index, W1, b1, W2, b2):
    row = edge_index[0].reshape(NW, NCH, CH)
    col = edge_index[1].reshape(NW, NCH, CH)
    attr = edge_attr[:, 0].reshape(NW, NCH, CH)
    ones_ch = jnp.ones((CH,), _f32)
    zrow = jnp.zeros((SPW,), _f32)
    zrow16 = jnp.zeros((SPW, HID), _f32)

    degp, xep = _sc_pass_a(row, col, attr, ones_ch, zrow)
    g1, dinv = _tc_pass_b(x, degp[0, :N], degp[1, :N], xep[0, :N], xep[1, :N], W1)
    s1p = _sc_pass_c(row, col, g1, zrow16)
    g2 = _tc_pass_d(s1p[0, :N, :], s1p[1, :N, :], g1, dinv, b1, W2)
    s2p = _sc_pass_e(row, col, g2, zrow)
    return _tc_pass_f(s2p[0, :N], s2p[1, :N], g2, dinv, b2)
